# baseline (speedup 1.0000x reference)
import jax
import jax.numpy as jnp
from jax import lax
from jax.experimental import pallas as pl
from jax.experimental.pallas import tpu as pltpu
from jax.experimental.pallas import tpu_sc as plsc

_B, _L, _D = 4, 4096, 1024
_CH = 16
_PAIR = 256
_NCH0, _NCH1 = 9, 7
_ST0, _ST1 = _NCH0 * _B, _NCH1 * _B
_XNB = 5
_PF = 3


def _sc_body(x_hbm, emb_hbm, p_hbm, out_hbm, idx_v, *rest):
    xb = rest[:_XNB]
    eb = rest[_XNB:_XNB + 2]
    xl_sem = rest[_XNB + 2:2 * _XNB + 2]
    st_sem = rest[2 * _XNB + 2:3 * _XNB + 2]
    eg_sem = rest[3 * _XNB + 2:]

    core = lax.axis_index("c")
    sub = lax.axis_index("s")
    is0 = core == 0
    lw0 = sub * _PAIR + core * (_NCH0 * _CH)

    @pl.when(is0)
    def _():
        pltpu.sync_copy(p_hbm.at[pl.ds(lw0, _NCH0 * _CH)], idx_v)

    @pl.when(jnp.logical_not(is0))
    def _():
        pltpu.sync_copy(p_hbm.at[pl.ds(lw0, _NCH1 * _CH)],
                        idx_v.at[pl.ds(0, _NCH1 * _CH)])

    def egather(c):
        return pltpu.make_async_copy(
            emb_hbm.at[idx_v.at[pl.ds(c * _CH, _CH)]], eb[c % 2],
            eg_sem[c % 2])

    def xcopy(t, store):
        c, b = t // _B, t % _B
        hbm_slice = out_hbm if store else x_hbm
        hbm_slice = hbm_slice.at[b, pl.ds(lw0 + c * _CH, _CH)]
        buf = xb[t % _XNB]
        sem = (st_sem if store else xl_sem)[t % _XNB]
        if store:
            return pltpu.make_async_copy(buf, hbm_slice, sem)
        return pltpu.make_async_copy(hbm_slice, buf, sem)

    def guarded(cond, fn):
        if cond is None:
            fn()
        else:
            pl.when(cond)(fn)

    egather(0).start()
    egather(1).start()
    for t in range(_PF):
        xcopy(t, False).start()

    for t in range(_ST0):
        c, b = t // _B, t % _B
        xs, es = t % _XNB, c % 2
        step_guard = None if t < _ST1 else is0

        def do_step(t=t, c=c, b=b, xs=xs, es=es):
            if b == 0:
                egather(c).wait()
            xcopy(t, False).wait()

            xbuf, ebuf = xb[xs], eb[es]

            @plsc.parallel_loop(0, _CH * _D // 16, 1, unroll=8)
            def _(i):
                r = i // (_D // 16)
                off = (i - r * (_D // 16)) * 16
                plsc.addupdate(xbuf.at[r, pl.ds(off, 16)],
                               ebuf[r, pl.ds(off, 16)])

            xcopy(t, True).start()

        guarded(step_guard, do_step)

        if b == _B - 1:
            if c + 2 < _NCH1:
                guarded(step_guard, lambda c=c: egather(c + 2).start())
            elif c + 2 < _NCH0:
                guarded(is0, lambda c=c: egather(c + 2).start())

        def prefetch(t=t):
            if t - 2 >= 0:
                xcopy(t - 2, True).wait()
            xcopy(t + _PF, False).start()

        if t + _PF < _ST1:
            guarded(step_guard, prefetch)
        elif t + _PF < _ST0:
            guarded(is0, prefetch)

    for t in range(_ST1 - _PF - 2, _ST1):
        guarded(jnp.logical_not(is0), lambda t=t: xcopy(t, True).wait())
    for t in range(_ST0 - _PF - 2, _ST0):
        guarded(is0, lambda t=t: xcopy(t, True).wait())


def kernel(X, embedding, offset):
    B, L, D = X.shape
    P = jnp.arange(L, dtype=jnp.int32) + jnp.asarray(offset, jnp.int32)
    f = pl.kernel(
        _sc_body,
        out_type=jax.ShapeDtypeStruct(X.shape, X.dtype),
        mesh=plsc.VectorSubcoreMesh(core_axis_name="c", subcore_axis_name="s"),
        scratch_types=[
            pltpu.VMEM((_NCH0 * _CH,), jnp.int32),
            *[pltpu.VMEM((_CH, D), jnp.float32) for _ in range(_XNB)],
            *[pltpu.VMEM((_CH, D), jnp.float32) for _ in range(2)],
            *[pltpu.SemaphoreType.DMA for _ in range(2 * _XNB + 2)],
        ],
    )
    return f(X, embedding, P)


# speedup vs baseline: 1.0222x; 1.0222x over previous
import jax
import jax.numpy as jnp
from jax import lax
from jax.experimental import pallas as pl
from jax.experimental.pallas import tpu as pltpu
from jax.experimental.pallas import tpu_sc as plsc

_B, _L, _D = 4, 4096, 1024
_CH = 16
_PAIR = 256
_NCH0, _NCH1 = 9, 7
_ST0, _ST1 = _NCH0 * _B, _NCH1 * _B
_XNB = 5
_PF = 3


def _sc_body(x_hbm, emb_hbm, p_hbm, out_hbm, idx_v, *rest):
    xb = rest[:_XNB]
    eb = rest[_XNB:_XNB + 2]
    xl_sem = rest[_XNB + 2:2 * _XNB + 2]
    st_sem = rest[2 * _XNB + 2:3 * _XNB + 2]
    eg_sem = rest[3 * _XNB + 2:]

    core = 1 - lax.axis_index("c")
    sub = lax.axis_index("s")
    is0 = core == 0
    lw0 = sub * _PAIR + core * (_NCH0 * _CH)

    @pl.when(is0)
    def _():
        pltpu.sync_copy(p_hbm.at[pl.ds(lw0, _NCH0 * _CH)], idx_v)

    @pl.when(jnp.logical_not(is0))
    def _():
        pltpu.sync_copy(p_hbm.at[pl.ds(lw0, _NCH1 * _CH)],
                        idx_v.at[pl.ds(0, _NCH1 * _CH)])

    def egather(c):
        return pltpu.make_async_copy(
            emb_hbm.at[idx_v.at[pl.ds(c * _CH, _CH)]], eb[c % 2],
            eg_sem[c % 2])

    def xcopy(t, store):
        c, b = t // _B, t % _B
        hbm_slice = out_hbm if store else x_hbm
        hbm_slice = hbm_slice.at[b, pl.ds(lw0 + c * _CH, _CH)]
        buf = xb[t % _XNB]
        sem = (st_sem if store else xl_sem)[t % _XNB]
        if store:
            return pltpu.make_async_copy(buf, hbm_slice, sem)
        return pltpu.make_async_copy(hbm_slice, buf, sem)

    def guarded(cond, fn):
        if cond is None:
            fn()
        else:
            pl.when(cond)(fn)

    egather(0).start()
    egather(1).start()
    for t in range(_PF):
        xcopy(t, False).start()

    for t in range(_ST0):
        c, b = t // _B, t % _B
        xs, es = t % _XNB, c % 2
        step_guard = None if t < _ST1 else is0

        def do_step(t=t, c=c, b=b, xs=xs, es=es):
            if b == 0:
                egather(c).wait()
            xcopy(t, False).wait()

            xbuf, ebuf = xb[xs], eb[es]

            @plsc.parallel_loop(0, _CH * _D // 16, 1, unroll=8)
            def _(i):
                r = i // (_D // 16)
                off = (i - r * (_D // 16)) * 16
                plsc.addupdate(xbuf.at[r, pl.ds(off, 16)],
                               ebuf[r, pl.ds(off, 16)])

            xcopy(t, True).start()

        guarded(step_guard, do_step)

        if b == _B - 1:
            if c + 2 < _NCH1:
                guarded(step_guard, lambda c=c: egather(c + 2).start())
            elif c + 2 < _NCH0:
                guarded(is0, lambda c=c: egather(c + 2).start())

        def prefetch(t=t):
            if t - 2 >= 0:
                xcopy(t - 2, True).wait()
            xcopy(t + _PF, False).start()

        if t + _PF < _ST1:
            guarded(step_guard, prefetch)
        elif t + _PF < _ST0:
            guarded(is0, prefetch)

    for t in range(_ST1 - _PF - 2, _ST1):
        guarded(jnp.logical_not(is0), lambda t=t: xcopy(t, True).wait())
    for t in range(_ST0 - _PF - 2, _ST0):
        guarded(is0, lambda t=t: xcopy(t, True).wait())


def kernel(X, embedding, offset):
    B, L, D = X.shape
    P = jnp.arange(L, dtype=jnp.int32) + jnp.asarray(offset, jnp.int32)
    f = pl.kernel(
        _sc_body,
        out_type=jax.ShapeDtypeStruct(X.shape, X.dtype),
        mesh=plsc.VectorSubcoreMesh(core_axis_name="c", subcore_axis_name="s"),
        scratch_types=[
            pltpu.VMEM((_NCH0 * _CH,), jnp.int32),
            *[pltpu.VMEM((_CH, D), jnp.float32) for _ in range(_XNB)],
            *[pltpu.VMEM((_CH, D), jnp.float32) for _ in range(2)],
            *[pltpu.SemaphoreType.DMA for _ in range(2 * _XNB + 2)],
        ],
    )
    return f(X, embedding, P)
